# baseline (speedup 1.0000x reference)
import jax
import jax.numpy as jnp
from jax import lax
from jax.experimental import pallas as pl
from jax.experimental.pallas import tpu as pltpu

N = 50000
D = 512
DH = 256
G = 256
B = 1024
NPAD = 50176
NB = NPAD // B


def _gate_pool_kernel(xb_ref, bb_ref, w1_ref, b1_ref, w2_ref, b2_ref,
                      s_out_ref, m_out_ref, d_out_ref, pooled_ref,
                      m_acc, d_acc, num_acc):
    i = pl.program_id(0)

    @pl.when(i == 0)
    def _init():
        m_acc[...] = jnp.full((1, G), -jnp.inf, jnp.float32)
        d_acc[...] = jnp.zeros((1, G), jnp.float32)
        num_acc[...] = jnp.zeros((G, D), jnp.float32)

    xb = xb_ref[...]
    h = jnp.tanh(
        jnp.dot(xb, w1_ref[...], preferred_element_type=jnp.float32)
        + b1_ref[...])
    s = (jnp.dot(h, w2_ref[...], preferred_element_type=jnp.float32)
         + b2_ref[...])
    s_out_ref[...] = s

    bb = bb_ref[...]
    ig = lax.broadcasted_iota(jnp.int32, (B, G), 1)
    oh = bb == ig

    bm = jnp.max(jnp.where(oh, s, -jnp.inf), axis=0, keepdims=True)
    m_old = m_acc[...]
    m_new = jnp.maximum(m_old, bm)
    alpha = jnp.where(m_old == -jnp.inf, 0.0, jnp.exp(m_old - m_new))

    mrow = jnp.sum(jnp.where(oh, m_new, 0.0), axis=1, keepdims=True)
    e = jnp.exp(s - mrow)
    ohe = jnp.where(oh, e, 0.0)
    d_add = jnp.sum(ohe, axis=0, keepdims=True)
    num_add = lax.dot_general(ohe.astype(jnp.bfloat16), xb,
                              (((0,), (0,)), ((), ())),
                              preferred_element_type=jnp.float32)

    eye = (lax.broadcasted_iota(jnp.int32, (G, G), 0)
           == lax.broadcasted_iota(jnp.int32, (G, G), 1))
    alpha_col = jnp.sum(jnp.where(eye, alpha, 0.0), axis=1, keepdims=True)

    d_acc[...] = d_acc[...] * alpha + d_add
    num_acc[...] = num_acc[...] * alpha_col + num_add
    m_acc[...] = m_new

    @pl.when(i == NB - 1)
    def _fin():
        d = d_acc[...]
        d_col = jnp.sum(jnp.where(eye, d, 0.0), axis=1, keepdims=True)
        pooled_ref[...] = jnp.where(d_col > 0, num_acc[...] / d_col, 0.0)
        m_out_ref[...] = m_acc[...]
        d_out_ref[...] = d


def _weights_kernel(s_ref, bb_ref, m_ref, d_ref, w_ref):
    s = s_ref[...]
    bb = bb_ref[...]
    ig = lax.broadcasted_iota(jnp.int32, (B, G), 1)
    oh = bb == ig
    mrow = jnp.sum(jnp.where(oh, m_ref[...], 0.0), axis=1, keepdims=True)
    drow = jnp.sum(jnp.where(oh, d_ref[...], 0.0), axis=1, keepdims=True)
    e = jnp.exp(s - mrow)
    w_ref[...] = jnp.where(drow > 0, e / drow, 0.0)


def kernel(x, batch, W1, b1, W2, b2):
    batch = batch.astype(jnp.int32)
    pad = NPAD - N
    xp = jnp.pad(x.astype(jnp.bfloat16), ((0, pad), (0, 0)))
    bp = jnp.pad(batch, (0, pad), constant_values=G).reshape(NPAD, 1)
    w1c = W1.astype(jnp.bfloat16)
    b1r = b1.reshape(1, DH).astype(jnp.float32)
    b2r = b2.reshape(1, 1).astype(jnp.float32)

    scores, m, d, pooled = pl.pallas_call(
        _gate_pool_kernel,
        grid=(NB,),
        in_specs=[
            pl.BlockSpec((B, D), lambda i: (i, 0)),
            pl.BlockSpec((B, 1), lambda i: (i, 0)),
            pl.BlockSpec((D, DH), lambda i: (0, 0)),
            pl.BlockSpec((1, DH), lambda i: (0, 0)),
            pl.BlockSpec((DH, 1), lambda i: (0, 0)),
            pl.BlockSpec((1, 1), lambda i: (0, 0)),
        ],
        out_specs=[
            pl.BlockSpec((B, 1), lambda i: (i, 0)),
            pl.BlockSpec((1, G), lambda i: (0, 0)),
            pl.BlockSpec((1, G), lambda i: (0, 0)),
            pl.BlockSpec((G, D), lambda i: (0, 0)),
        ],
        out_shape=[
            jax.ShapeDtypeStruct((NPAD, 1), jnp.float32),
            jax.ShapeDtypeStruct((1, G), jnp.float32),
            jax.ShapeDtypeStruct((1, G), jnp.float32),
            jax.ShapeDtypeStruct((G, D), jnp.float32),
        ],
        scratch_shapes=[
            pltpu.VMEM((1, G), jnp.float32),
            pltpu.VMEM((1, G), jnp.float32),
            pltpu.VMEM((G, D), jnp.float32),
        ],
    )(xp, bp, w1c, b1r, W2.astype(jnp.float32), b2r)

    weights = pl.pallas_call(
        _weights_kernel,
        grid=(NB,),
        in_specs=[
            pl.BlockSpec((B, 1), lambda i: (i, 0)),
            pl.BlockSpec((B, 1), lambda i: (i, 0)),
            pl.BlockSpec((1, G), lambda i: (0, 0)),
            pl.BlockSpec((1, G), lambda i: (0, 0)),
        ],
        out_specs=pl.BlockSpec((B, 1), lambda i: (i, 0)),
        out_shape=jax.ShapeDtypeStruct((NPAD, 1), jnp.float32),
    )(scores, bp, m, d)

    return (pooled, weights[:N, 0])


# speedup vs baseline: 1.6061x; 1.6061x over previous
import jax
import jax.numpy as jnp
from jax import lax
from jax.experimental import pallas as pl
from jax.experimental.pallas import tpu as pltpu

N = 50000
D = 512
DH = 256
G = 256
B = 2048
NB = (N + B - 1) // B


def _gate_pool_kernel(xb_ref, bb_ref, w1_ref, b1_ref, w2_ref, b2_ref,
                      s_out_ref, m_out_ref, d_out_ref, pooled_ref,
                      m_acc, d_acc, num_acc):
    i = pl.program_id(0)

    @pl.when(i == 0)
    def _init():
        m_acc[...] = jnp.full((1, G), -jnp.inf, jnp.float32)
        d_acc[...] = jnp.zeros((1, G), jnp.float32)
        num_acc[...] = jnp.zeros((G, D), jnp.float32)

    xb = xb_ref[...]
    h = jnp.tanh(
        jnp.dot(xb, w1_ref[...], preferred_element_type=jnp.float32)
        + b1_ref[...])
    s = (jnp.dot(h, w2_ref[...], preferred_element_type=jnp.float32)
         + b2_ref[...])
    s_out_ref[...] = s

    row = i * B + lax.broadcasted_iota(jnp.int32, (B, 1), 0)
    valid = row < N
    bb = bb_ref[...]
    ig = lax.broadcasted_iota(jnp.int32, (B, G), 1)
    oh = (bb == ig) & valid

    bm = jnp.max(jnp.where(oh, s, -jnp.inf), axis=0, keepdims=True)
    m_old = m_acc[...]
    m_new = jnp.maximum(m_old, bm)
    alpha = jnp.where(m_old == -jnp.inf, 0.0, jnp.exp(m_old - m_new))

    mrow = jnp.sum(jnp.where(oh, m_new, 0.0), axis=1, keepdims=True)
    e = jnp.exp(s - mrow)
    ohe = jnp.where(oh, e, 0.0)
    d_add = jnp.sum(ohe, axis=0, keepdims=True)
    xv = jnp.where(valid, xb, 0.0)
    num_add = lax.dot_general(ohe, xv, (((0,), (0,)), ((), ())),
                              preferred_element_type=jnp.float32)

    eye = (lax.broadcasted_iota(jnp.int32, (G, G), 0)
           == lax.broadcasted_iota(jnp.int32, (G, G), 1))
    alpha_col = jnp.sum(jnp.where(eye, alpha, 0.0), axis=1, keepdims=True)

    d_acc[...] = d_acc[...] * alpha + d_add
    num_acc[...] = num_acc[...] * alpha_col + num_add
    m_acc[...] = m_new

    @pl.when(i == NB - 1)
    def _fin():
        d = d_acc[...]
        d_col = jnp.sum(jnp.where(eye, d, 0.0), axis=1, keepdims=True)
        pooled_ref[...] = jnp.where(d_col > 0, num_acc[...] / d_col, 0.0)
        m_out_ref[...] = m_acc[...]
        d_out_ref[...] = d


def _weights_kernel(s_ref, bb_ref, m_ref, d_ref, w_ref):
    s = s_ref[...]
    bb = bb_ref[...]
    ig = lax.broadcasted_iota(jnp.int32, (B, G), 1)
    oh = bb == ig
    mrow = jnp.sum(jnp.where(oh, m_ref[...], 0.0), axis=1, keepdims=True)
    drow = jnp.sum(jnp.where(oh, d_ref[...], 0.0), axis=1, keepdims=True)
    e = jnp.exp(s - mrow)
    w_ref[...] = jnp.where(drow > 0, e / drow, 0.0)


def kernel(x, batch, W1, b1, W2, b2):
    x = x.astype(jnp.float32)
    bp = batch.astype(jnp.int32).reshape(N, 1)
    b1r = b1.reshape(1, DH).astype(jnp.float32)
    b2r = b2.reshape(1, 1).astype(jnp.float32)

    scores, m, d, pooled = pl.pallas_call(
        _gate_pool_kernel,
        grid=(NB,),
        in_specs=[
            pl.BlockSpec((B, D), lambda i: (i, 0)),
            pl.BlockSpec((B, 1), lambda i: (i, 0)),
            pl.BlockSpec((D, DH), lambda i: (0, 0)),
            pl.BlockSpec((1, DH), lambda i: (0, 0)),
            pl.BlockSpec((DH, 1), lambda i: (0, 0)),
            pl.BlockSpec((1, 1), lambda i: (0, 0)),
        ],
        out_specs=[
            pl.BlockSpec((B, 1), lambda i: (i, 0)),
            pl.BlockSpec((1, G), lambda i: (0, 0)),
            pl.BlockSpec((1, G), lambda i: (0, 0)),
            pl.BlockSpec((G, D), lambda i: (0, 0)),
        ],
        out_shape=[
            jax.ShapeDtypeStruct((N, 1), jnp.float32),
            jax.ShapeDtypeStruct((1, G), jnp.float32),
            jax.ShapeDtypeStruct((1, G), jnp.float32),
            jax.ShapeDtypeStruct((G, D), jnp.float32),
        ],
        scratch_shapes=[
            pltpu.VMEM((1, G), jnp.float32),
            pltpu.VMEM((1, G), jnp.float32),
            pltpu.VMEM((G, D), jnp.float32),
        ],
    )(x, bp, W1.astype(jnp.float32), b1r, W2.astype(jnp.float32), b2r)

    weights = pl.pallas_call(
        _weights_kernel,
        grid=(NB,),
        in_specs=[
            pl.BlockSpec((B, 1), lambda i: (i, 0)),
            pl.BlockSpec((B, 1), lambda i: (i, 0)),
            pl.BlockSpec((1, G), lambda i: (0, 0)),
            pl.BlockSpec((1, G), lambda i: (0, 0)),
        ],
        out_specs=pl.BlockSpec((B, 1), lambda i: (i, 0)),
        out_shape=jax.ShapeDtypeStruct((N, 1), jnp.float32),
    )(scores, bp, m, d)

    return (pooled, weights[:, 0])
